# Initial kernel scaffold; baseline (speedup 1.0000x reference)
import jax
import jax.numpy as jnp
from jax.experimental import pallas as pl


def kernel(logits, y_true):
    raise NotImplementedError("write your pallas kernel here")


# speedup vs baseline: 3.4347x; 3.4347x over previous
import functools

import jax
import jax.numpy as jnp
from jax.experimental import pallas as pl

_C = 1000
_BK = 512


def _loss_block(logits_ref, y_ref, out_ref, *, num_rows):
    i = pl.program_id(0)

    x = logits_ref[...]
    y = y_ref[0, 0, :]

    m = jnp.max(x, axis=1, keepdims=True)
    lse = jnp.log(jnp.sum(jnp.exp(x - m), axis=1)) + m[:, 0]

    col = jax.lax.broadcasted_iota(jnp.int32, x.shape, 1)
    yb = y[:, None]
    c32 = jnp.int32(_C)
    one = jnp.int32(1)
    prev = jax.lax.rem(yb - one + c32, c32)
    nxt = jax.lax.rem(yb + one, c32)
    w = (
        jnp.where(col == yb, 0.8, 0.0)
        + jnp.where(col == prev, 0.1, 0.0)
        + jnp.where(col == nxt, 0.1, 0.0)
    )
    dot = jnp.sum(w * x, axis=1)

    partial = (jnp.sum(lse - dot) / num_rows).reshape(1, 1)

    @pl.when(i == 0)
    def _():
        out_ref[...] = partial

    @pl.when(i != 0)
    def _():
        out_ref[...] += partial


def kernel(logits, y_true):
    b, c = logits.shape
    y = y_true.astype(jnp.int32).reshape(b // _BK, 1, _BK)
    grid = b // _BK
    out = pl.pallas_call(
        functools.partial(_loss_block, num_rows=b),
        grid=(grid,),
        in_specs=[
            pl.BlockSpec((_BK, c), lambda i: (i, 0)),
            pl.BlockSpec((1, 1, _BK), lambda i: (i, 0, 0)),
        ],
        out_specs=pl.BlockSpec((1, 1), lambda i: (0, 0)),
        out_shape=jax.ShapeDtypeStruct((1, 1), jnp.float32),
    )(logits, y)
    return out[0, 0]
